# baseline (speedup 1.0000x reference)
import functools
import numpy as np
import jax
import jax.numpy as jnp
from jax import lax
from jax.experimental import pallas as pl
from jax.experimental.pallas import tpu as pltpu
from jax.experimental.pallas import tpu_sc as plsc

N = 8192
NLANE = 16
NVREG = N // NLANE
HIST_SZ = 4608
INT_MIN32 = jnp.int32(-(2**31))

_QUANTILES = np.float32([0.1, 0.25, 0.5, 0.75, 0.9])


def _const_lanes(vals, dtype):
    io = lax.iota(jnp.int32, NLANE)
    v = jnp.full((NLANE,), dtype(0), dtype)
    for i, x in enumerate(vals):
        v = jnp.where(io == i, dtype(x), v)
    return v


def _shuffle(tmp_ref, v, idx):
    tmp_ref[...] = v
    return plsc.load_gather(tmp_ref, [idx])


def _quantile_rows_sc(x, ranks, w_lo, w_hi):
    rows = x.shape[0]
    info = plsc.get_sparse_core_info()
    nw = info.num_cores * info.num_subcores
    rpw = rows // nw
    mesh = plsc.VectorSubcoreMesh(core_axis_name="c", subcore_axis_name="s")

    rank10 = []
    for k in ranks:
        rank10 += [int(k), int(k) + 1]

    @functools.partial(
        pl.kernel, mesh=mesh,
        out_type=jax.ShapeDtypeStruct((rows, NLANE), jnp.float32),
        scratch_types=[
            pltpu.VMEM((N,), jnp.float32),
            pltpu.VMEM((N,), jnp.int32),
            pltpu.VMEM((HIST_SZ,), jnp.int32),
            pltpu.VMEM((256,), jnp.int32),
            pltpu.VMEM((4096,), jnp.int32),
            pltpu.VMEM((4096,), jnp.int32),
            pltpu.VMEM((rpw, NLANE), jnp.float32),
            pltpu.VMEM((NLANE,), jnp.int32),
            pltpu.VMEM((NLANE,), jnp.float32),
        ],
        compiler_params=pltpu.CompilerParams(needs_layout_passes=False),
    )
    def qkernel(x_hbm, out_hbm, xbuf, keybuf, hist, map1, map2, map3, outbuf,
                tmp16i, tmp16f):
        wid = lax.axis_index("s") * info.num_cores + lax.axis_index("c")
        base = wid * rpw
        io = lax.iota(jnp.int32, NLANE)
        ones = jnp.ones((NLANE,), jnp.int32)
        zeros16 = jnp.zeros((NLANE,), jnp.int32)
        live = io < 10
        kinit = _const_lanes(rank10, jnp.int32)
        prev_idx = jnp.maximum(io - 1, 0)

        def clear(ref, nwords):
            @plsc.parallel_loop(0, nwords // NLANE, unroll=16)
            def _cb(j):
                ref[pl.ds(j * NLANE, NLANE)] = zeros16

        def row_body(rr, _carry):
            pltpu.sync_copy(x_hbm.at[base + rr], xbuf)
            clear(hist, HIST_SZ)

            @plsc.parallel_loop(0, NVREG, unroll=16)
            def _l1(i):
                xv = xbuf[pl.ds(i * NLANE, NLANE)]
                iv = lax.bitcast_convert_type(xv, jnp.int32)
                ukey = iv ^ (lax.shift_right_arithmetic(iv, 31) | INT_MIN32)
                keybuf[pl.ds(i * NLANE, NLANE)] = ukey
                d1 = lax.shift_right_logical(ukey, 24)
                idx = lax.shift_left(d1, 4) + io
                plsc.addupdate_scatter(hist, [idx], ones)

            def w1(j, c):
                cum, dig, bs = c
                hv = hist[pl.ds(j * NLANE, NLANE)]
                cum = cum + jnp.sum(hv)
                le = cum <= kadj0
                dig = dig + jnp.where(le, 1, 0)
                bs = jnp.where(le, cum, bs)
                return (cum, dig, bs)
            cum0 = jnp.int32(0)
            _, dig1, base1 = plsc.parallel_loop(
                0, 256, carry=(cum0, zeros16, zeros16), unroll=16)(w1)
            kadj1 = kadj0 - base1
            ufound1 = lax.shift_left(dig1, 24)

            bnd = (dig1 != _shuffle(tmp16i, dig1, prev_idx)) | (io == 0)
            grp1 = plsc.cumsum(jnp.where(bnd, 1, 0)) - 1
            m1_idx = jnp.minimum(dig1, 255)
            m1_mask = bnd & live
            plsc.store_scatter(map1, [m1_idx], grp1 + 1, mask=m1_mask)
            clear(hist, HIST_SZ)

            @plsc.parallel_loop(0, NVREG, unroll=16)
            def _l2(i):
                u = keybuf[pl.ds(i * NLANE, NLANE)]
                d1 = lax.shift_right_logical(u, 24)
                g = plsc.load_gather(map1, [d1])
                comp = lax.shift_left(g, 24) | (u & jnp.int32(0xFFFFFF))
                keybuf[pl.ds(i * NLANE, NLANE)] = comp
                plsc.addupdate_scatter(
                    hist, [lax.shift_right_logical(comp, 16)], ones)
            plsc.store_scatter(map1, [m1_idx], zeros16, mask=m1_mask)

            def walk(level_grp, kadj):
                gbase = lax.shift_left(level_grp + 1, 8)
                def wb(j, c):
                    cum, dig, bs = c
                    cnt = plsc.load_gather(hist, [gbase + j])
                    cum = cum + cnt
                    le = cum <= kadj
                    dig = dig + jnp.where(le, 1, 0)
                    bs = jnp.where(le, cum, bs)
                    return (cum, dig, bs)
                _, dig, bs = plsc.parallel_loop(
                    0, 256, carry=(zeros16, zeros16, zeros16), unroll=16)(wb)
                return dig, bs

            def regroup(old_grp, dig, mref):
                pk = lax.shift_left(old_grp, 12) | dig
                bnd2 = (pk != _shuffle(tmp16i, pk, prev_idx)) | (io == 0)
                ng = plsc.cumsum(jnp.where(bnd2, 1, 0)) - 1
                idx = lax.shift_left(old_grp + 1, 8) + jnp.minimum(dig, 255)
                msk = bnd2 & live
                plsc.store_scatter(mref, [idx], ng + 1, mask=msk)
                return ng, idx, msk

            dig2, base2 = walk(grp1, kadj1)
            kadj2 = kadj1 - base2
            ufound2 = ufound1 | lax.shift_left(dig2, 16)
            grp2, m2_idx, m2_mask = regroup(grp1, dig2, map2)
            clear(hist, HIST_SZ)

            @plsc.parallel_loop(0, NVREG, unroll=16)
            def _l3(i):
                comp = keybuf[pl.ds(i * NLANE, NLANE)]
                pidx = lax.shift_right_logical(comp, 16)
                g = plsc.load_gather(map2, [pidx])
                comp2 = lax.shift_left(g, 16) | (comp & jnp.int32(0xFFFF))
                keybuf[pl.ds(i * NLANE, NLANE)] = comp2
                plsc.addupdate_scatter(
                    hist, [lax.shift_right_logical(comp2, 8)], ones)
            plsc.store_scatter(map2, [m2_idx], zeros16, mask=m2_mask)

            dig3, base3 = walk(grp2, kadj2)
            kadj3 = kadj2 - base3
            ufound3 = ufound2 | lax.shift_left(dig3, 8)
            grp3, m3_idx, m3_mask = regroup(grp2, dig3, map3)
            clear(hist, HIST_SZ)

            @plsc.parallel_loop(0, NVREG, unroll=16)
            def _l4(i):
                comp = keybuf[pl.ds(i * NLANE, NLANE)]
                pidx = lax.shift_right_logical(comp, 8)
                g = plsc.load_gather(map3, [pidx])
                idx = lax.shift_left(g, 8) | (comp & jnp.int32(0xFF))
                plsc.addupdate_scatter(hist, [idx], ones)
            plsc.store_scatter(map3, [m3_idx], zeros16, mask=m3_mask)

            dig4, _b4 = walk(grp3, kadj3)
            ukey = ufound3 | dig4

            key = ukey ^ INT_MIN32
            iv = jnp.where(key >= 0, key, key ^ jnp.int32(0x7FFFFFFF))
            f = lax.bitcast_convert_type(iv, jnp.float32)
            tmp16f[...] = f
            flo = plsc.load_gather(tmp16f, [jnp.minimum(io * 2, 15)])
            fhi = plsc.load_gather(tmp16f, [jnp.minimum(io * 2 + 1, 15)])
            outv = flo * wlo_v + fhi * whi_v
            outbuf[rr] = outv
            return 0

        kadj0 = kinit
        wlo_v = _const_lanes(list(w_lo), jnp.float32)
        whi_v = _const_lanes(list(w_hi), jnp.float32)
        clear(map1, 256)
        clear(map2, 4096)
        clear(map3, 4096)
        lax.fori_loop(0, rpw, row_body, 0)
        pltpu.sync_copy(outbuf, out_hbm.at[pl.ds(base, rpw)])

    return qkernel(x)


def kernel(input):
    b, t, n = input.shape
    rows = b * t
    x = input.reshape(rows, n)
    idxf = _QUANTILES * np.float32(n - 1)
    ranks = np.floor(idxf).astype(np.int32)
    w_hi = (idxf - ranks).astype(np.float32)
    w_lo = (np.float32(1.0) - w_hi).astype(np.float32)
    out = _quantile_rows_sc(x, ranks, w_lo, w_hi)
    return out[:, :5].reshape(b, t, 5)


# speedup vs baseline: 1.4670x; 1.4670x over previous
import functools
import numpy as np
import jax
import jax.numpy as jnp
from jax import lax
from jax.experimental import pallas as pl
from jax.experimental.pallas import tpu as pltpu
from jax.experimental.pallas import tpu_sc as plsc

N = 8192
NLANE = 16
NVREG = N // NLANE
HIST_SZ = 4608
INT_MIN32 = jnp.int32(-(2**31))

_QUANTILES = np.float32([0.1, 0.25, 0.5, 0.75, 0.9])


def _const_lanes(vals, dtype):
    io = lax.iota(jnp.int32, NLANE)
    v = jnp.full((NLANE,), dtype(0), dtype)
    for i, x in enumerate(vals):
        v = jnp.where(io == i, dtype(x), v)
    return v


def _shuffle(tmp_ref, v, idx):
    tmp_ref[...] = v
    return plsc.load_gather(tmp_ref, [idx])


def _quantile_rows_sc(x, ranks, w_lo, w_hi):
    rows = x.shape[0]
    info = plsc.get_sparse_core_info()
    nw = info.num_cores * info.num_subcores
    rpw = rows // nw
    mesh = plsc.VectorSubcoreMesh(core_axis_name="c", subcore_axis_name="s")

    rank10 = []
    for k in ranks:
        rank10 += [int(k), int(k) + 1]

    @functools.partial(
        pl.kernel, mesh=mesh,
        out_type=jax.ShapeDtypeStruct((rows, NLANE), jnp.float32),
        scratch_types=[
            pltpu.VMEM((N,), jnp.float32),
            pltpu.VMEM((N,), jnp.int32),
            pltpu.VMEM((HIST_SZ,), jnp.int32),
            pltpu.VMEM((256,), jnp.int32),
            pltpu.VMEM((4096,), jnp.int32),
            pltpu.VMEM((4096,), jnp.int32),
            pltpu.VMEM((rpw, NLANE), jnp.float32),
            pltpu.VMEM((NLANE,), jnp.int32),
            pltpu.VMEM((NLANE,), jnp.float32),
        ],
        compiler_params=pltpu.CompilerParams(needs_layout_passes=False),
    )
    def qkernel(x_hbm, out_hbm, xbuf, keybuf, hist, map1, map2, map3, outbuf,
                tmp16i, tmp16f):
        wid = lax.axis_index("s") * info.num_cores + lax.axis_index("c")
        base = wid * rpw
        io = lax.iota(jnp.int32, NLANE)
        ones = jnp.ones((NLANE,), jnp.int32)
        zeros16 = jnp.zeros((NLANE,), jnp.int32)
        live = io < 10
        kinit = _const_lanes(rank10, jnp.int32)
        prev_idx = jnp.maximum(io - 1, 0)

        def clear(ref, nwords):
            @plsc.parallel_loop(0, nwords // NLANE, unroll=8)
            def _cb(j):
                ref[pl.ds(j * NLANE, NLANE)] = zeros16

        def row_body(rr, _carry):
            pltpu.sync_copy(x_hbm.at[base + rr], xbuf)
            clear(hist, HIST_SZ)

            @plsc.parallel_loop(0, NVREG, unroll=8)
            def _l1(i):
                xv = xbuf[pl.ds(i * NLANE, NLANE)]
                iv = lax.bitcast_convert_type(xv, jnp.int32)
                ukey = iv ^ (lax.shift_right_arithmetic(iv, 31) | INT_MIN32)
                keybuf[pl.ds(i * NLANE, NLANE)] = ukey
                d1 = lax.shift_right_logical(ukey, 24)
                idx = lax.shift_left(d1, 4) + io
                plsc.addupdate_scatter(hist, [idx], ones)

            def w1(j, c):
                cum, dig, bs = c
                hv = hist[pl.ds(j * NLANE, NLANE)]
                cum = cum + jnp.sum(hv)
                le = cum <= kadj0
                dig = dig + jnp.where(le, 1, 0)
                bs = jnp.where(le, cum, bs)
                return (cum, dig, bs)
            cum0 = jnp.int32(0)
            _, dig1, base1 = plsc.parallel_loop(
                0, 256, carry=(cum0, zeros16, zeros16), unroll=8)(w1)
            kadj1 = kadj0 - base1
            ufound1 = lax.shift_left(dig1, 24)

            bnd = (dig1 != _shuffle(tmp16i, dig1, prev_idx)) | (io == 0)
            grp1 = plsc.cumsum(jnp.where(bnd, 1, 0)) - 1
            m1_idx = jnp.minimum(dig1, 255)
            m1_mask = bnd & live
            plsc.store_scatter(map1, [m1_idx], grp1 + 1, mask=m1_mask)
            clear(hist, HIST_SZ)

            @plsc.parallel_loop(0, NVREG, unroll=8)
            def _l2(i):
                u = keybuf[pl.ds(i * NLANE, NLANE)]
                d1 = lax.shift_right_logical(u, 24)
                g = plsc.load_gather(map1, [d1])
                comp = lax.shift_left(g, 24) | (u & jnp.int32(0xFFFFFF))
                keybuf[pl.ds(i * NLANE, NLANE)] = comp
                plsc.addupdate_scatter(
                    hist, [lax.shift_right_logical(comp, 16)], ones)
            plsc.store_scatter(map1, [m1_idx], zeros16, mask=m1_mask)

            def walk(level_grp, kadj):
                gbase = lax.shift_left(level_grp + 1, 8)
                def wb(j, c):
                    cum, dig, bs = c
                    cnt = plsc.load_gather(hist, [gbase + j])
                    cum = cum + cnt
                    le = cum <= kadj
                    dig = dig + jnp.where(le, 1, 0)
                    bs = jnp.where(le, cum, bs)
                    return (cum, dig, bs)
                _, dig, bs = plsc.parallel_loop(
                    0, 256, carry=(zeros16, zeros16, zeros16), unroll=8)(wb)
                return dig, bs

            def regroup(old_grp, dig, mref):
                pk = lax.shift_left(old_grp, 12) | dig
                bnd2 = (pk != _shuffle(tmp16i, pk, prev_idx)) | (io == 0)
                ng = plsc.cumsum(jnp.where(bnd2, 1, 0)) - 1
                idx = lax.shift_left(old_grp + 1, 8) + jnp.minimum(dig, 255)
                msk = bnd2 & live
                plsc.store_scatter(mref, [idx], ng + 1, mask=msk)
                return ng, idx, msk

            dig2, base2 = walk(grp1, kadj1)
            kadj2 = kadj1 - base2
            ufound2 = ufound1 | lax.shift_left(dig2, 16)
            grp2, m2_idx, m2_mask = regroup(grp1, dig2, map2)
            clear(hist, HIST_SZ)

            @plsc.parallel_loop(0, NVREG, unroll=8)
            def _l3(i):
                comp = keybuf[pl.ds(i * NLANE, NLANE)]
                pidx = lax.shift_right_logical(comp, 16)
                g = plsc.load_gather(map2, [pidx])
                comp2 = lax.shift_left(g, 16) | (comp & jnp.int32(0xFFFF))
                keybuf[pl.ds(i * NLANE, NLANE)] = comp2
                plsc.addupdate_scatter(
                    hist, [lax.shift_right_logical(comp2, 8)], ones)
            plsc.store_scatter(map2, [m2_idx], zeros16, mask=m2_mask)

            dig3, base3 = walk(grp2, kadj2)
            kadj3 = kadj2 - base3
            ufound3 = ufound2 | lax.shift_left(dig3, 8)
            grp3, m3_idx, m3_mask = regroup(grp2, dig3, map3)
            clear(hist, HIST_SZ)

            @plsc.parallel_loop(0, NVREG, unroll=8)
            def _l4(i):
                comp = keybuf[pl.ds(i * NLANE, NLANE)]
                pidx = lax.shift_right_logical(comp, 8)
                g = plsc.load_gather(map3, [pidx])
                idx = lax.shift_left(g, 8) | (comp & jnp.int32(0xFF))
                plsc.addupdate_scatter(hist, [idx], ones)
            plsc.store_scatter(map3, [m3_idx], zeros16, mask=m3_mask)

            dig4, _b4 = walk(grp3, kadj3)
            ukey = ufound3 | dig4

            key = ukey ^ INT_MIN32
            iv = jnp.where(key >= 0, key, key ^ jnp.int32(0x7FFFFFFF))
            f = lax.bitcast_convert_type(iv, jnp.float32)
            tmp16f[...] = f
            flo = plsc.load_gather(tmp16f, [jnp.minimum(io * 2, 15)])
            fhi = plsc.load_gather(tmp16f, [jnp.minimum(io * 2 + 1, 15)])
            outv = flo * wlo_v + fhi * whi_v
            outbuf[rr] = outv
            return 0

        kadj0 = kinit
        wlo_v = _const_lanes(list(w_lo), jnp.float32)
        whi_v = _const_lanes(list(w_hi), jnp.float32)
        clear(map1, 256)
        clear(map2, 4096)
        clear(map3, 4096)
        lax.fori_loop(0, rpw, row_body, 0)
        pltpu.sync_copy(outbuf, out_hbm.at[pl.ds(base, rpw)])

    return qkernel(x)


_NQ = 5
_INT_MIN = np.int32(-(2**31))
_INT_MAX = np.int32(2**31 - 1)


def _quantile_body(ranks, w_lo, w_hi, x_ref, o_ref):
    x = x_ref[...]
    i = jax.lax.bitcast_convert_type(x, jnp.int32)
    key = i ^ (jax.lax.shift_right_arithmetic(i, 31) & jnp.int32(0x7FFFFFFF))

    br = x.shape[0]

    def step(it, r):
        bit = jax.lax.shift_left(jnp.int32(1), jnp.int32(31) - it)
        t = r + bit
        p = _INT_MIN + t
        m01 = (jnp.where(key < p[:, 0:1], 1, 0)
               + jnp.where(key < p[:, 1:2], 1 << 16, 0))
        m23 = (jnp.where(key < p[:, 2:3], 1, 0)
               + jnp.where(key < p[:, 3:4], 1 << 16, 0))
        m4 = jnp.where(key < p[:, 4:5], 1, 0)
        s01 = jnp.sum(m01, axis=1, keepdims=True)
        s23 = jnp.sum(m23, axis=1, keepdims=True)
        s4 = jnp.sum(m4, axis=1, keepdims=True)
        counts = [s01 & 0xFFFF, jax.lax.shift_right_logical(s01, 16),
                  s23 & 0xFFFF, jax.lax.shift_right_logical(s23, 16), s4]
        cols = [jnp.where(counts[q] <= int(ranks[q]), t[:, q:q + 1],
                          r[:, q:q + 1]) for q in range(_NQ)]
        return jnp.concatenate(cols, axis=1)

    r = jax.lax.fori_loop(0, 32, step, jnp.zeros((br, _NQ), jnp.int32))
    rkey = _INT_MIN + r

    outs = []
    for q in range(_NQ):
        lo = rkey[:, q:q + 1]
        le = jnp.sum((key <= lo).astype(jnp.int32), axis=1, keepdims=True)
        gt_min = jnp.min(jnp.where(key > lo, key, _INT_MAX), axis=1,
                         keepdims=True)
        hi = jnp.where(le >= ranks[q] + 2, lo, gt_min)
        f_lo = _key_to_f32(lo)
        f_hi = _key_to_f32(hi)
        outs.append(f_lo * w_lo[q] + f_hi * w_hi[q])
    o_ref[...] = jnp.concatenate(outs, axis=1)


def _key_to_f32(key):
    i = jnp.where(key >= 0, key, key ^ jnp.int32(0x7FFFFFFF))
    return jax.lax.bitcast_convert_type(i, jnp.float32)


def _quantile_rows_tc(x, ranks, w_lo, w_hi, br):
    rows, n = x.shape
    grid = rows // br
    return pl.pallas_call(
        functools.partial(_quantile_body, ranks, w_lo, w_hi),
        grid=(grid,),
        in_specs=[pl.BlockSpec((br, n), lambda g: (g, 0))],
        out_specs=pl.BlockSpec((br, _NQ), lambda g: (g, 0)),
        out_shape=jax.ShapeDtypeStruct((rows, _NQ), jnp.float32),
        compiler_params=pltpu.CompilerParams(
            dimension_semantics=("arbitrary",),
        ),
    )(x)


def kernel(input):
    b, t, n = input.shape
    rows = b * t
    x = input.reshape(rows, n)
    idxf = _QUANTILES * np.float32(n - 1)
    ranks = np.floor(idxf).astype(np.int32)
    w_hi = (idxf - ranks).astype(np.float32)
    w_lo = (np.float32(1.0) - w_hi).astype(np.float32)

    rows_sc = (int(rows * 0.63) // 128) * 128
    rows_tc = rows - rows_sc
    if rows_tc == 0 or rows_tc % 8 != 0:
        rows_sc, rows_tc = rows, 0
    out_sc = _quantile_rows_sc(x[:rows_sc], ranks, w_lo, w_hi)[:, :5]
    if rows_tc:
        br = 128 if rows_tc % 128 == 0 else 8
        out_tc = _quantile_rows_tc(x[rows_sc:], ranks, w_lo, w_hi, br)
        out = jnp.concatenate([out_sc, out_tc], axis=0)
    else:
        out = out_sc
    return out.reshape(b, t, 5)


# speedup vs baseline: 1.4675x; 1.0003x over previous
import functools
import numpy as np
import jax
import jax.numpy as jnp
from jax import lax
from jax.experimental import pallas as pl
from jax.experimental.pallas import tpu as pltpu
from jax.experimental.pallas import tpu_sc as plsc

N = 8192
NLANE = 16
NVREG = N // NLANE
HIST_SZ = 4608
INT_MIN32 = jnp.int32(-(2**31))

_QUANTILES = np.float32([0.1, 0.25, 0.5, 0.75, 0.9])


def _const_lanes(vals, dtype):
    io = lax.iota(jnp.int32, NLANE)
    v = jnp.full((NLANE,), dtype(0), dtype)
    for i, x in enumerate(vals):
        v = jnp.where(io == i, dtype(x), v)
    return v


def _shuffle(tmp_ref, v, idx):
    tmp_ref[...] = v
    return plsc.load_gather(tmp_ref, [idx])


def _quantile_rows_sc(x, ranks, w_lo, w_hi):
    rows = x.shape[0]
    info = plsc.get_sparse_core_info()
    nw = info.num_cores * info.num_subcores
    rpw = rows // nw
    mesh = plsc.VectorSubcoreMesh(core_axis_name="c", subcore_axis_name="s")

    rank10 = []
    for k in ranks:
        rank10 += [int(k), int(k) + 1]

    @functools.partial(
        pl.kernel, mesh=mesh,
        out_type=jax.ShapeDtypeStruct((rows, NLANE), jnp.float32),
        scratch_types=[
            pltpu.VMEM((N,), jnp.float32),
            pltpu.VMEM((N,), jnp.float32),
            pltpu.VMEM((N,), jnp.int32),
            pltpu.VMEM((HIST_SZ,), jnp.int32),
            pltpu.VMEM((256,), jnp.int32),
            pltpu.VMEM((4096,), jnp.int32),
            pltpu.VMEM((4096,), jnp.int32),
            pltpu.VMEM((rpw, NLANE), jnp.float32),
            pltpu.VMEM((NLANE,), jnp.int32),
            pltpu.VMEM((NLANE,), jnp.float32),
            pltpu.SemaphoreType.DMA,
            pltpu.SemaphoreType.DMA,
        ],
        compiler_params=pltpu.CompilerParams(needs_layout_passes=False),
    )
    def qkernel(x_hbm, out_hbm, xbuf0, xbuf1, keybuf, hist, map1, map2, map3,
                outbuf, tmp16i, tmp16f, sem0, sem1):
        wid = lax.axis_index("s") * info.num_cores + lax.axis_index("c")
        base = wid * rpw
        io = lax.iota(jnp.int32, NLANE)
        ones = jnp.ones((NLANE,), jnp.int32)
        zeros16 = jnp.zeros((NLANE,), jnp.int32)
        live = io < 10
        kinit = _const_lanes(rank10, jnp.int32)
        prev_idx = jnp.maximum(io - 1, 0)

        def clear(ref, nwords):
            @plsc.parallel_loop(0, nwords // NLANE, unroll=8)
            def _cb(j):
                ref[pl.ds(j * NLANE, NLANE)] = zeros16

        def row_body(rr, xbuf):
            clear(hist, HIST_SZ)

            @plsc.parallel_loop(0, NVREG, unroll=8)
            def _l1(i):
                xv = xbuf[pl.ds(i * NLANE, NLANE)]
                iv = lax.bitcast_convert_type(xv, jnp.int32)
                ukey = iv ^ (lax.shift_right_arithmetic(iv, 31) | INT_MIN32)
                keybuf[pl.ds(i * NLANE, NLANE)] = ukey
                d1 = lax.shift_right_logical(ukey, 24)
                idx = lax.shift_left(d1, 4) + io
                plsc.addupdate_scatter(hist, [idx], ones)

            def w1(j, c):
                cum, dig, bs = c
                hv = hist[pl.ds(j * NLANE, NLANE)]
                cum = cum + jnp.sum(hv)
                le = cum <= kadj0
                dig = dig + jnp.where(le, 1, 0)
                bs = jnp.where(le, cum, bs)
                return (cum, dig, bs)
            cum0 = jnp.int32(0)
            _, dig1, base1 = plsc.parallel_loop(
                0, 256, carry=(cum0, zeros16, zeros16), unroll=8)(w1)
            kadj1 = kadj0 - base1
            ufound1 = lax.shift_left(dig1, 24)

            bnd = (dig1 != _shuffle(tmp16i, dig1, prev_idx)) | (io == 0)
            grp1 = plsc.cumsum(jnp.where(bnd, 1, 0)) - 1
            m1_idx = jnp.minimum(dig1, 255)
            m1_mask = bnd & live
            plsc.store_scatter(map1, [m1_idx], grp1 + 1, mask=m1_mask)
            clear(hist, HIST_SZ)

            @plsc.parallel_loop(0, NVREG, unroll=8)
            def _l2(i):
                u = keybuf[pl.ds(i * NLANE, NLANE)]
                d1 = lax.shift_right_logical(u, 24)
                g = plsc.load_gather(map1, [d1])
                comp = lax.shift_left(g, 24) | (u & jnp.int32(0xFFFFFF))
                keybuf[pl.ds(i * NLANE, NLANE)] = comp
                plsc.addupdate_scatter(
                    hist, [lax.shift_right_logical(comp, 16)], ones)
            plsc.store_scatter(map1, [m1_idx], zeros16, mask=m1_mask)

            def walk(level_grp, kadj):
                gbase = lax.shift_left(level_grp + 1, 8)
                def wb(j, c):
                    cum, dig, bs = c
                    cnt = plsc.load_gather(hist, [gbase + j])
                    cum = cum + cnt
                    le = cum <= kadj
                    dig = dig + jnp.where(le, 1, 0)
                    bs = jnp.where(le, cum, bs)
                    return (cum, dig, bs)
                _, dig, bs = plsc.parallel_loop(
                    0, 256, carry=(zeros16, zeros16, zeros16), unroll=8)(wb)
                return dig, bs

            def regroup(old_grp, dig, mref):
                pk = lax.shift_left(old_grp, 12) | dig
                bnd2 = (pk != _shuffle(tmp16i, pk, prev_idx)) | (io == 0)
                ng = plsc.cumsum(jnp.where(bnd2, 1, 0)) - 1
                idx = lax.shift_left(old_grp + 1, 8) + jnp.minimum(dig, 255)
                msk = bnd2 & live
                plsc.store_scatter(mref, [idx], ng + 1, mask=msk)
                return ng, idx, msk

            dig2, base2 = walk(grp1, kadj1)
            kadj2 = kadj1 - base2
            ufound2 = ufound1 | lax.shift_left(dig2, 16)
            grp2, m2_idx, m2_mask = regroup(grp1, dig2, map2)
            clear(hist, HIST_SZ)

            @plsc.parallel_loop(0, NVREG, unroll=8)
            def _l3(i):
                comp = keybuf[pl.ds(i * NLANE, NLANE)]
                pidx = lax.shift_right_logical(comp, 16)
                g = plsc.load_gather(map2, [pidx])
                comp2 = lax.shift_left(g, 16) | (comp & jnp.int32(0xFFFF))
                keybuf[pl.ds(i * NLANE, NLANE)] = comp2
                plsc.addupdate_scatter(
                    hist, [lax.shift_right_logical(comp2, 8)], ones)
            plsc.store_scatter(map2, [m2_idx], zeros16, mask=m2_mask)

            dig3, base3 = walk(grp2, kadj2)
            kadj3 = kadj2 - base3
            ufound3 = ufound2 | lax.shift_left(dig3, 8)
            grp3, m3_idx, m3_mask = regroup(grp2, dig3, map3)
            clear(hist, HIST_SZ)

            @plsc.parallel_loop(0, NVREG, unroll=8)
            def _l4(i):
                comp = keybuf[pl.ds(i * NLANE, NLANE)]
                pidx = lax.shift_right_logical(comp, 8)
                g = plsc.load_gather(map3, [pidx])
                idx = lax.shift_left(g, 8) | (comp & jnp.int32(0xFF))
                plsc.addupdate_scatter(hist, [idx], ones)
            plsc.store_scatter(map3, [m3_idx], zeros16, mask=m3_mask)

            dig4, _b4 = walk(grp3, kadj3)
            ukey = ufound3 | dig4

            key = ukey ^ INT_MIN32
            iv = jnp.where(key >= 0, key, key ^ jnp.int32(0x7FFFFFFF))
            f = lax.bitcast_convert_type(iv, jnp.float32)
            tmp16f[...] = f
            flo = plsc.load_gather(tmp16f, [jnp.minimum(io * 2, 15)])
            fhi = plsc.load_gather(tmp16f, [jnp.minimum(io * 2 + 1, 15)])
            outv = flo * wlo_v + fhi * whi_v
            outbuf[rr] = outv
            return 0

        kadj0 = kinit
        wlo_v = _const_lanes(list(w_lo), jnp.float32)
        whi_v = _const_lanes(list(w_hi), jnp.float32)
        clear(map1, 256)
        clear(map2, 4096)
        clear(map3, 4096)

        last = rows - 1
        pltpu.make_async_copy(x_hbm.at[base], xbuf0, sem0).start()
        pltpu.make_async_copy(x_hbm.at[base + 1], xbuf1, sem1).start()

        def pair_body(i, _):
            r0 = 2 * i
            pltpu.make_async_copy(x_hbm.at[base + r0], xbuf0, sem0).wait()
            row_body(r0, xbuf0)
            nxt0 = jnp.minimum(base + r0 + 2, last)
            pltpu.make_async_copy(x_hbm.at[nxt0], xbuf0, sem0).start()
            pltpu.make_async_copy(x_hbm.at[base + r0 + 1], xbuf1, sem1).wait()
            row_body(r0 + 1, xbuf1)
            nxt1 = jnp.minimum(base + r0 + 3, last)
            pltpu.make_async_copy(x_hbm.at[nxt1], xbuf1, sem1).start()
            return 0
        lax.fori_loop(0, rpw // 2, pair_body, 0)
        pltpu.make_async_copy(x_hbm.at[last], xbuf0, sem0).wait()
        pltpu.make_async_copy(x_hbm.at[last], xbuf1, sem1).wait()
        pltpu.sync_copy(outbuf, out_hbm.at[pl.ds(base, rpw)])

    return qkernel(x)


_NQ = 5
_INT_MIN = np.int32(-(2**31))
_INT_MAX = np.int32(2**31 - 1)


def _quantile_body(ranks, w_lo, w_hi, x_ref, o_ref):
    x = x_ref[...]
    i = jax.lax.bitcast_convert_type(x, jnp.int32)
    key = i ^ (jax.lax.shift_right_arithmetic(i, 31) & jnp.int32(0x7FFFFFFF))

    br = x.shape[0]

    def step(it, r):
        bit = jax.lax.shift_left(jnp.int32(1), jnp.int32(31) - it)
        t = r + bit
        p = _INT_MIN + t
        m01 = (jnp.where(key < p[:, 0:1], 1, 0)
               + jnp.where(key < p[:, 1:2], 1 << 16, 0))
        m23 = (jnp.where(key < p[:, 2:3], 1, 0)
               + jnp.where(key < p[:, 3:4], 1 << 16, 0))
        m4 = jnp.where(key < p[:, 4:5], 1, 0)
        s01 = jnp.sum(m01, axis=1, keepdims=True)
        s23 = jnp.sum(m23, axis=1, keepdims=True)
        s4 = jnp.sum(m4, axis=1, keepdims=True)
        counts = [s01 & 0xFFFF, jax.lax.shift_right_logical(s01, 16),
                  s23 & 0xFFFF, jax.lax.shift_right_logical(s23, 16), s4]
        cols = [jnp.where(counts[q] <= int(ranks[q]), t[:, q:q + 1],
                          r[:, q:q + 1]) for q in range(_NQ)]
        return jnp.concatenate(cols, axis=1)

    r = jax.lax.fori_loop(0, 32, step, jnp.zeros((br, _NQ), jnp.int32))
    rkey = _INT_MIN + r

    outs = []
    for q in range(_NQ):
        lo = rkey[:, q:q + 1]
        le = jnp.sum((key <= lo).astype(jnp.int32), axis=1, keepdims=True)
        gt_min = jnp.min(jnp.where(key > lo, key, _INT_MAX), axis=1,
                         keepdims=True)
        hi = jnp.where(le >= ranks[q] + 2, lo, gt_min)
        f_lo = _key_to_f32(lo)
        f_hi = _key_to_f32(hi)
        outs.append(f_lo * w_lo[q] + f_hi * w_hi[q])
    o_ref[...] = jnp.concatenate(outs, axis=1)


def _key_to_f32(key):
    i = jnp.where(key >= 0, key, key ^ jnp.int32(0x7FFFFFFF))
    return jax.lax.bitcast_convert_type(i, jnp.float32)


def _quantile_rows_tc(x, ranks, w_lo, w_hi, br):
    rows, n = x.shape
    grid = rows // br
    return pl.pallas_call(
        functools.partial(_quantile_body, ranks, w_lo, w_hi),
        grid=(grid,),
        in_specs=[pl.BlockSpec((br, n), lambda g: (g, 0))],
        out_specs=pl.BlockSpec((br, _NQ), lambda g: (g, 0)),
        out_shape=jax.ShapeDtypeStruct((rows, _NQ), jnp.float32),
        compiler_params=pltpu.CompilerParams(
            dimension_semantics=("arbitrary",),
        ),
    )(x)


def kernel(input):
    b, t, n = input.shape
    rows = b * t
    x = input.reshape(rows, n)
    idxf = _QUANTILES * np.float32(n - 1)
    ranks = np.floor(idxf).astype(np.int32)
    w_hi = (idxf - ranks).astype(np.float32)
    w_lo = (np.float32(1.0) - w_hi).astype(np.float32)

    rows_sc = (int(rows * 0.63) // 128) * 128
    rows_tc = rows - rows_sc
    if rows_tc == 0 or rows_tc % 8 != 0:
        rows_sc, rows_tc = rows, 0
    out_sc = _quantile_rows_sc(x[:rows_sc], ranks, w_lo, w_hi)[:, :5]
    if rows_tc:
        br = 128 if rows_tc % 128 == 0 else 8
        out_tc = _quantile_rows_tc(x[rows_sc:], ranks, w_lo, w_hi, br)
        out = jnp.concatenate([out_sc, out_tc], axis=0)
    else:
        out = out_sc
    return out.reshape(b, t, 5)


# speedup vs baseline: 1.5877x; 1.0819x over previous
import functools
import numpy as np
import jax
import jax.numpy as jnp
from jax import lax
from jax.experimental import pallas as pl
from jax.experimental.pallas import tpu as pltpu
from jax.experimental.pallas import tpu_sc as plsc

N = 8192
NLANE = 16
NVREG = N // NLANE
HIST_SZ = 4608
INT_MIN32 = jnp.int32(-(2**31))

_QUANTILES = np.float32([0.1, 0.25, 0.5, 0.75, 0.9])


def _const_lanes(vals, dtype):
    io = lax.iota(jnp.int32, NLANE)
    v = jnp.full((NLANE,), dtype(0), dtype)
    for i, x in enumerate(vals):
        v = jnp.where(io == i, dtype(x), v)
    return v


def _shuffle(tmp_ref, v, idx):
    tmp_ref[...] = v
    return plsc.load_gather(tmp_ref, [idx])


def _quantile_rows_sc(x, ranks, w_lo, w_hi):
    rows = x.shape[0]
    info = plsc.get_sparse_core_info()
    nw = info.num_cores * info.num_subcores
    rpw = rows // nw
    mesh = plsc.VectorSubcoreMesh(core_axis_name="c", subcore_axis_name="s")

    rank10 = []
    for k in ranks:
        rank10 += [int(k), int(k) + 1]

    @functools.partial(
        pl.kernel, mesh=mesh,
        out_type=jax.ShapeDtypeStruct((rows, NLANE), jnp.float32),
        scratch_types=[
            pltpu.VMEM((N,), jnp.float32),
            pltpu.VMEM((N,), jnp.float32),
            pltpu.VMEM((N,), jnp.int32),
            pltpu.VMEM((HIST_SZ,), jnp.int32),
            pltpu.VMEM((256,), jnp.int32),
            pltpu.VMEM((4096,), jnp.int32),
            pltpu.VMEM((4096,), jnp.int32),
            pltpu.VMEM((rpw, NLANE), jnp.float32),
            pltpu.VMEM((NLANE,), jnp.int32),
            pltpu.VMEM((NLANE,), jnp.float32),
            pltpu.SemaphoreType.DMA,
            pltpu.SemaphoreType.DMA,
        ],
        compiler_params=pltpu.CompilerParams(needs_layout_passes=False),
    )
    def qkernel(x_hbm, out_hbm, xbuf0, xbuf1, keybuf, hist, map1, map2, map3,
                outbuf, tmp16i, tmp16f, sem0, sem1):
        wid = lax.axis_index("s") * info.num_cores + lax.axis_index("c")
        base = wid * rpw
        io = lax.iota(jnp.int32, NLANE)
        ones = jnp.ones((NLANE,), jnp.int32)
        zeros16 = jnp.zeros((NLANE,), jnp.int32)
        live = io < 10
        kinit = _const_lanes(rank10, jnp.int32)
        prev_idx = jnp.maximum(io - 1, 0)

        def clear(ref, nwords):
            @plsc.parallel_loop(0, nwords // NLANE, unroll=8)
            def _cb(j):
                ref[pl.ds(j * NLANE, NLANE)] = zeros16

        def row_body(rr, xbuf):
            clear(hist, HIST_SZ)

            @plsc.parallel_loop(0, NVREG, unroll=8)
            def _l1(i):
                xv = xbuf[pl.ds(i * NLANE, NLANE)]
                iv = lax.bitcast_convert_type(xv, jnp.int32)
                ukey = iv ^ (lax.shift_right_arithmetic(iv, 31) | INT_MIN32)
                keybuf[pl.ds(i * NLANE, NLANE)] = ukey
                d1 = lax.shift_right_logical(ukey, 24)
                idx = lax.shift_left(d1, 4) + io
                plsc.addupdate_scatter(hist, [idx], ones)

            def w1(j, c):
                cum, dig, bs = c
                hv = hist[pl.ds(j * NLANE, NLANE)]
                cum = cum + jnp.sum(hv)
                le = cum <= kadj0
                dig = dig + jnp.where(le, 1, 0)
                bs = jnp.where(le, cum, bs)
                return (cum, dig, bs)
            cum0 = jnp.int32(0)
            _, dig1, base1 = plsc.parallel_loop(
                0, 256, carry=(cum0, zeros16, zeros16), unroll=8)(w1)
            kadj1 = kadj0 - base1
            ufound1 = lax.shift_left(dig1, 24)

            bnd = (dig1 != _shuffle(tmp16i, dig1, prev_idx)) | (io == 0)
            grp1 = plsc.cumsum(jnp.where(bnd, 1, 0)) - 1
            m1_idx = jnp.minimum(dig1, 255)
            m1_mask = bnd & live
            plsc.store_scatter(map1, [m1_idx], grp1 + 1, mask=m1_mask)
            clear(hist, HIST_SZ)

            @plsc.parallel_loop(0, NVREG, unroll=8)
            def _l2(i):
                u = keybuf[pl.ds(i * NLANE, NLANE)]
                d1 = lax.shift_right_logical(u, 24)
                g = plsc.load_gather(map1, [d1])
                comp = lax.shift_left(g, 24) | (u & jnp.int32(0xFFFFFF))
                keybuf[pl.ds(i * NLANE, NLANE)] = comp
                plsc.addupdate_scatter(
                    hist, [lax.shift_right_logical(comp, 16)], ones)
            plsc.store_scatter(map1, [m1_idx], zeros16, mask=m1_mask)

            def walk(level_grp, kadj):
                gbase = lax.shift_left(level_grp + 1, 8)
                def wb(j, c):
                    cum, dig, bs = c
                    cnt = plsc.load_gather(hist, [gbase + j])
                    cum = cum + cnt
                    le = cum <= kadj
                    dig = dig + jnp.where(le, 1, 0)
                    bs = jnp.where(le, cum, bs)
                    return (cum, dig, bs)
                _, dig, bs = plsc.parallel_loop(
                    0, 256, carry=(zeros16, zeros16, zeros16), unroll=8)(wb)
                return dig, bs

            def regroup(old_grp, dig, mref):
                pk = lax.shift_left(old_grp, 12) | dig
                bnd2 = (pk != _shuffle(tmp16i, pk, prev_idx)) | (io == 0)
                ng = plsc.cumsum(jnp.where(bnd2, 1, 0)) - 1
                idx = lax.shift_left(old_grp + 1, 8) + jnp.minimum(dig, 255)
                msk = bnd2 & live
                plsc.store_scatter(mref, [idx], ng + 1, mask=msk)
                return ng, idx, msk

            dig2, base2 = walk(grp1, kadj1)
            kadj2 = kadj1 - base2
            ufound2 = ufound1 | lax.shift_left(dig2, 16)
            grp2, m2_idx, m2_mask = regroup(grp1, dig2, map2)
            clear(hist, HIST_SZ)

            @plsc.parallel_loop(0, NVREG, unroll=8)
            def _l3(i):
                comp = keybuf[pl.ds(i * NLANE, NLANE)]
                pidx = lax.shift_right_logical(comp, 16)
                g = plsc.load_gather(map2, [pidx])
                comp2 = lax.shift_left(g, 16) | (comp & jnp.int32(0xFFFF))
                keybuf[pl.ds(i * NLANE, NLANE)] = comp2
                plsc.addupdate_scatter(
                    hist, [lax.shift_right_logical(comp2, 8)], ones)
            plsc.store_scatter(map2, [m2_idx], zeros16, mask=m2_mask)

            dig3, base3 = walk(grp2, kadj2)
            kadj3 = kadj2 - base3
            ufound3 = ufound2 | lax.shift_left(dig3, 8)
            grp3, m3_idx, m3_mask = regroup(grp2, dig3, map3)
            clear(hist, HIST_SZ)

            @plsc.parallel_loop(0, NVREG, unroll=8)
            def _l4(i):
                comp = keybuf[pl.ds(i * NLANE, NLANE)]
                pidx = lax.shift_right_logical(comp, 8)
                g = plsc.load_gather(map3, [pidx])
                idx = lax.shift_left(g, 8) | (comp & jnp.int32(0xFF))
                plsc.addupdate_scatter(hist, [idx], ones)
            plsc.store_scatter(map3, [m3_idx], zeros16, mask=m3_mask)

            dig4, _b4 = walk(grp3, kadj3)
            ukey = ufound3 | dig4

            key = ukey ^ INT_MIN32
            iv = jnp.where(key >= 0, key, key ^ jnp.int32(0x7FFFFFFF))
            f = lax.bitcast_convert_type(iv, jnp.float32)
            tmp16f[...] = f
            flo = plsc.load_gather(tmp16f, [jnp.minimum(io * 2, 15)])
            fhi = plsc.load_gather(tmp16f, [jnp.minimum(io * 2 + 1, 15)])
            outv = flo * wlo_v + fhi * whi_v
            outbuf[rr] = outv
            return 0

        kadj0 = kinit
        wlo_v = _const_lanes(list(w_lo), jnp.float32)
        whi_v = _const_lanes(list(w_hi), jnp.float32)
        clear(map1, 256)
        clear(map2, 4096)
        clear(map3, 4096)

        last = rows - 1
        pltpu.make_async_copy(x_hbm.at[base], xbuf0, sem0).start()
        pltpu.make_async_copy(x_hbm.at[base + 1], xbuf1, sem1).start()

        def pair_body(i, _):
            r0 = 2 * i
            pltpu.make_async_copy(x_hbm.at[base + r0], xbuf0, sem0).wait()
            row_body(r0, xbuf0)
            nxt0 = jnp.minimum(base + r0 + 2, last)
            pltpu.make_async_copy(x_hbm.at[nxt0], xbuf0, sem0).start()
            pltpu.make_async_copy(x_hbm.at[base + r0 + 1], xbuf1, sem1).wait()
            row_body(r0 + 1, xbuf1)
            nxt1 = jnp.minimum(base + r0 + 3, last)
            pltpu.make_async_copy(x_hbm.at[nxt1], xbuf1, sem1).start()
            return 0
        lax.fori_loop(0, rpw // 2, pair_body, 0)
        pltpu.make_async_copy(x_hbm.at[last], xbuf0, sem0).wait()
        pltpu.make_async_copy(x_hbm.at[last], xbuf1, sem1).wait()
        pltpu.sync_copy(outbuf, out_hbm.at[pl.ds(base, rpw)])

    return qkernel(x)


_NQ = 5
_INT_MIN = np.int32(-(2**31))
_INT_MAX = np.int32(2**31 - 1)


def _quantile_body(ranks, w_lo, w_hi, x_ref, o_ref):
    x = x_ref[...]
    i = jax.lax.bitcast_convert_type(x, jnp.int32)
    key = i ^ (jax.lax.shift_right_arithmetic(i, 31) & jnp.int32(0x7FFFFFFF))

    br = x.shape[0]

    def step(it, r):
        bit = jax.lax.shift_left(jnp.int32(1), jnp.int32(31) - it)
        t = r + bit
        p = _INT_MIN + t
        m01 = (jnp.where(key < p[:, 0:1], 1, 0)
               + jnp.where(key < p[:, 1:2], 1 << 16, 0))
        m23 = (jnp.where(key < p[:, 2:3], 1, 0)
               + jnp.where(key < p[:, 3:4], 1 << 16, 0))
        m4 = jnp.where(key < p[:, 4:5], 1, 0)
        s01 = jnp.sum(m01, axis=1, keepdims=True)
        s23 = jnp.sum(m23, axis=1, keepdims=True)
        s4 = jnp.sum(m4, axis=1, keepdims=True)
        counts = [s01 & 0xFFFF, jax.lax.shift_right_logical(s01, 16),
                  s23 & 0xFFFF, jax.lax.shift_right_logical(s23, 16), s4]
        cols = [jnp.where(counts[q] <= int(ranks[q]), t[:, q:q + 1],
                          r[:, q:q + 1]) for q in range(_NQ)]
        return jnp.concatenate(cols, axis=1)

    r = jax.lax.fori_loop(0, 32, step, jnp.zeros((br, _NQ), jnp.int32))
    rkey = _INT_MIN + r

    outs = []
    for q in range(_NQ):
        lo = rkey[:, q:q + 1]
        le = jnp.sum((key <= lo).astype(jnp.int32), axis=1, keepdims=True)
        gt_min = jnp.min(jnp.where(key > lo, key, _INT_MAX), axis=1,
                         keepdims=True)
        hi = jnp.where(le >= ranks[q] + 2, lo, gt_min)
        f_lo = _key_to_f32(lo)
        f_hi = _key_to_f32(hi)
        outs.append(f_lo * w_lo[q] + f_hi * w_hi[q])
    o_ref[...] = jnp.concatenate(outs, axis=1)


def _key_to_f32(key):
    i = jnp.where(key >= 0, key, key ^ jnp.int32(0x7FFFFFFF))
    return jax.lax.bitcast_convert_type(i, jnp.float32)


def _quantile_rows_tc(x, ranks, w_lo, w_hi, br):
    rows, n = x.shape
    grid = rows // br
    return pl.pallas_call(
        functools.partial(_quantile_body, ranks, w_lo, w_hi),
        grid=(grid,),
        in_specs=[pl.BlockSpec((br, n), lambda g: (g, 0))],
        out_specs=pl.BlockSpec((br, _NQ), lambda g: (g, 0)),
        out_shape=jax.ShapeDtypeStruct((rows, _NQ), jnp.float32),
        compiler_params=pltpu.CompilerParams(
            dimension_semantics=("arbitrary",),
        ),
    )(x)


def kernel(input):
    b, t, n = input.shape
    rows = b * t
    x = input.reshape(rows, n)
    idxf = _QUANTILES * np.float32(n - 1)
    ranks = np.floor(idxf).astype(np.int32)
    w_hi = (idxf - ranks).astype(np.float32)
    w_lo = (np.float32(1.0) - w_hi).astype(np.float32)

    rows_sc = (int(rows * 0.657) // 256) * 256
    rows_tc = rows - rows_sc
    if rows_tc == 0 or rows_tc % 8 != 0:
        rows_sc, rows_tc = rows, 0
    out_sc = _quantile_rows_sc(x[:rows_sc], ranks, w_lo, w_hi)[:, :5]
    if rows_tc:
        br = 128 if rows_tc % 128 == 0 else 8
        out_tc = _quantile_rows_tc(x[rows_sc:], ranks, w_lo, w_hi, br)
        out = jnp.concatenate([out_sc, out_tc], axis=0)
    else:
        out = out_sc
    return out.reshape(b, t, 5)
